# Initial kernel scaffold; baseline (speedup 1.0000x reference)
import jax
import jax.numpy as jnp
from jax.experimental import pallas as pl


def kernel(x, word_table, pos_table):
    raise NotImplementedError("write your pallas kernel here")


# speedup vs baseline: 2.0223x; 2.0223x over previous
import jax
import jax.numpy as jnp
from jax import lax
from jax.experimental import pallas as pl
from jax.experimental.pallas import tpu as pltpu
from jax.experimental.pallas import tpu_sc as plsc

_NC = 2
_NS = 16
_NW = _NC * _NS
_LANES = 16


def _make_sc_kernel(Bsz, Lsz, V, D, seq_per_w, cs, g_sub):
    rc = cs * Lsz
    nsub = rc // g_sub
    nch = seq_per_w // cs
    rows_per_w = seq_per_w * Lsz

    mesh = plsc.VectorSubcoreMesh(core_axis_name="c", subcore_axis_name="s")

    def body(idx_hbm, tab_hbm, pos_hbm, out_hbm, idx_v, pos_v, buf, gsem):
        c = lax.axis_index("c")
        s = lax.axis_index("s")
        wid = s * _NC + c
        pltpu.sync_copy(idx_hbm.at[pl.ds(wid * nch * nsub, nch * nsub)], idx_v)
        pltpu.sync_copy(pos_hbm, pos_v)
        row_base0 = wid * rows_per_w

        def chunk_body(g, carry):
            handles = []
            for k in range(nsub):
                h = pltpu.async_copy(
                    tab_hbm.at[idx_v.at[g * nsub + k]],
                    buf.at[pl.ds(k * g_sub, g_sub)],
                    gsem,
                )
                handles.append(h)
            for h in handles:
                h.wait()

            def row_body(r, carry2):
                l = lax.rem(r, Lsz)
                for cg in range(D // _LANES):
                    sl = pl.ds(cg * _LANES, _LANES)
                    buf[r, sl] = buf[r, sl] + pos_v[l, sl]
                return carry2

            lax.fori_loop(0, rc, row_body, None)

            pltpu.sync_copy(buf, out_hbm.at[pl.ds(row_base0 + g * rc, rc)])
            return carry

        lax.fori_loop(0, nch, chunk_body, None)

    return pl.kernel(
        body,
        out_type=jax.ShapeDtypeStruct((Bsz * Lsz, D), jnp.float32),
        mesh=mesh,
        compiler_params=pltpu.CompilerParams(use_tc_tiling_on_sc=False),
        scratch_types=[
            pltpu.VMEM((nch * nsub, g_sub), jnp.int32),
            pltpu.VMEM((Lsz, D), jnp.float32),
            pltpu.VMEM((rc, D), jnp.float32),
            pltpu.SemaphoreType.DMA,
        ],
    )


def kernel(x, word_table, pos_table):
    Bsz, Lsz = x.shape
    V, D = word_table.shape
    seq_per_w = Bsz // _NW
    cs = 2
    g_sub = 80

    idx = x.reshape(_NW * (seq_per_w // cs) * (cs * Lsz // g_sub), g_sub)
    idx = idx.astype(jnp.int32)
    pos_rows = pos_table[1 : Lsz + 1]
    sc = _make_sc_kernel(Bsz, Lsz, V, D, seq_per_w, cs, g_sub)
    out = sc(idx, word_table, pos_rows)
    return out.reshape(Bsz, Lsz, D)


# speedup vs baseline: 2.6633x; 1.3170x over previous
import jax
import jax.numpy as jnp
from jax import lax
from jax.experimental import pallas as pl
from jax.experimental.pallas import tpu as pltpu
from jax.experimental.pallas import tpu_sc as plsc

_NC = 2
_NS = 16
_NW = _NC * _NS
_LANES = 16


def _make_sc_kernel(Bsz, Lsz, V, D, seq_per_w, cs, g_sub):
    rc = cs * Lsz
    nsub = rc // g_sub
    nch = seq_per_w // cs
    rows_per_w = seq_per_w * Lsz

    mesh = plsc.VectorSubcoreMesh(core_axis_name="c", subcore_axis_name="s")

    def body(idx_hbm, tab_hbm, pos_hbm, out_hbm,
             idx_v, pos_v, buf_a, buf_b, gsem_a, gsem_b, osem_a, osem_b):
        c = lax.axis_index("c")
        s = lax.axis_index("s")
        wid = s * _NC + c
        pltpu.sync_copy(idx_hbm.at[pl.ds(wid * nch * nsub, nch * nsub)], idx_v)
        pltpu.sync_copy(pos_hbm, pos_v)
        row_base0 = wid * rows_per_w

        def fire_gathers(buf, sem, ch):
            return [
                pltpu.async_copy(
                    tab_hbm.at[idx_v.at[ch * nsub + k]],
                    buf.at[pl.ds(k * g_sub, g_sub)],
                    sem,
                )
                for k in range(nsub)
            ]

        def add_pos(buf):
            for sq in range(cs):
                base = sq * Lsz

                @plsc.parallel_loop(0, Lsz, unroll=4)
                def _(r):
                    for cg in range(D // _LANES):
                        sl = pl.ds(cg * _LANES, _LANES)
                        buf[base + r, sl] = buf[base + r, sl] + pos_v[r, sl]

        def store_out(buf, sem, ch):
            return pltpu.async_copy(
                buf, out_hbm.at[pl.ds(row_base0 + ch * rc, rc)], sem
            )

        def pair_body(g2, carry):
            ch_a = g2 * 2
            ch_b = ch_a + 1
            hs_a = fire_gathers(buf_a, gsem_a, ch_a)
            hs_b = fire_gathers(buf_b, gsem_b, ch_b)
            for h in hs_a:
                h.wait()
            add_pos(buf_a)
            out_a = store_out(buf_a, osem_a, ch_a)
            for h in hs_b:
                h.wait()
            add_pos(buf_b)
            out_b = store_out(buf_b, osem_b, ch_b)
            out_a.wait()
            out_b.wait()
            return carry

        lax.fori_loop(0, nch // 2, pair_body, None)

    return pl.kernel(
        body,
        out_type=jax.ShapeDtypeStruct((Bsz * Lsz, D), jnp.float32),
        mesh=mesh,
        compiler_params=pltpu.CompilerParams(use_tc_tiling_on_sc=False),
        scratch_types=[
            pltpu.VMEM((nch * nsub, g_sub), jnp.int32),
            pltpu.VMEM((Lsz, D), jnp.float32),
            pltpu.VMEM((rc, D), jnp.float32),
            pltpu.VMEM((rc, D), jnp.float32),
            pltpu.SemaphoreType.DMA,
            pltpu.SemaphoreType.DMA,
            pltpu.SemaphoreType.DMA,
            pltpu.SemaphoreType.DMA,
        ],
    )


def kernel(x, word_table, pos_table):
    Bsz, Lsz = x.shape
    V, D = word_table.shape
    seq_per_w = Bsz // _NW
    cs = 2
    g_sub = 80

    idx = x.reshape(_NW * (seq_per_w // cs) * (cs * Lsz // g_sub), g_sub)
    idx = idx.astype(jnp.int32)
    pos_rows = pos_table[1 : Lsz + 1]
    sc = _make_sc_kernel(Bsz, Lsz, V, D, seq_per_w, cs, g_sub)
    out = sc(idx, word_table, pos_rows)
    return out.reshape(Bsz, Lsz, D)
